# Initial kernel scaffold; baseline (speedup 1.0000x reference)
import jax
import jax.numpy as jnp
from jax.experimental import pallas as pl


def kernel(input_embeds, pos_table):
    raise NotImplementedError("write your pallas kernel here")


# speedup vs baseline: 2.9030x; 2.9030x over previous
import jax
import jax.numpy as jnp
from jax.experimental import pallas as pl


def _add_body(in_ref, pos_ref, out_ref):
    out_ref[...] = in_ref[...] + pos_ref[...]


def kernel(input_embeds, pos_table):
    batch, seq, d = input_embeds.shape
    S_BLK = 512
    grid = (seq // S_BLK, batch)

    return pl.pallas_call(
        _add_body,
        grid=grid,
        in_specs=[
            pl.BlockSpec((1, S_BLK, d), lambda s, b: (b, s, 0)),
            pl.BlockSpec((S_BLK, d), lambda s, b: (s, 0)),
        ],
        out_specs=pl.BlockSpec((1, S_BLK, d), lambda s, b: (b, s, 0)),
        out_shape=jax.ShapeDtypeStruct((batch, seq, d), input_embeds.dtype),
    )(input_embeds, pos_table)


# speedup vs baseline: 3.6227x; 1.2479x over previous
import jax
import jax.numpy as jnp
from jax.experimental import pallas as pl


def _add_body(in_ref, pos_ref, out_ref):
    out_ref[...] = in_ref[...] + pos_ref[...]


def kernel(input_embeds, pos_table):
    batch, seq, d = input_embeds.shape
    S_BLK = 2048
    grid = (seq // S_BLK, batch)

    return pl.pallas_call(
        _add_body,
        grid=grid,
        in_specs=[
            pl.BlockSpec((1, S_BLK, d), lambda s, b: (b, s, 0)),
            pl.BlockSpec((S_BLK, d), lambda s, b: (s, 0)),
        ],
        out_specs=pl.BlockSpec((1, S_BLK, d), lambda s, b: (b, s, 0)),
        out_shape=jax.ShapeDtypeStruct((batch, seq, d), input_embeds.dtype),
    )(input_embeds, pos_table)


# speedup vs baseline: 3.6321x; 1.0026x over previous
import jax
import jax.numpy as jnp
from jax.experimental import pallas as pl


def _add_body(in_ref, pos_ref, out_ref):
    out_ref[...] = in_ref[...] + pos_ref[...]


def kernel(input_embeds, pos_table):
    batch, seq, d = input_embeds.shape
    S_BLK = 1024
    B_BLK = 4
    grid = (seq // S_BLK, batch // B_BLK)

    return pl.pallas_call(
        _add_body,
        grid=grid,
        in_specs=[
            pl.BlockSpec((B_BLK, S_BLK, d), lambda s, b: (b, s, 0)),
            pl.BlockSpec((S_BLK, d), lambda s, b: (s, 0)),
        ],
        out_specs=pl.BlockSpec((B_BLK, S_BLK, d), lambda s, b: (b, s, 0)),
        out_shape=jax.ShapeDtypeStruct((batch, seq, d), input_embeds.dtype),
    )(input_embeds, pos_table)
